# baseline (speedup 1.0000x reference)
import functools

import jax
import jax.numpy as jnp
from jax import lax
from jax.experimental import pallas as pl
from jax.experimental.pallas import tpu as pltpu
from jax.experimental.pallas import tpu_sc as plsc

_N = 100000
_E = 6400000
_LN = 256
_ROWS = _E // _LN
_KB = 5
_NC = 2
_NS = 16
_NPAD1 = 100096
_ST1 = _NPAD1 // _NS
_NPAD2 = 100096
_ST2 = _NPAD2 // _NS
_CK2 = 184
_NCK2 = _ST2 // _CK2


def _mesh():
    return plsc.VectorSubcoreMesh(
        core_axis_name="c", subcore_axis_name="s",
        num_cores=_NC, num_subcores=_NS)


def _tile_range(total, s):
    per, rem = total // _NS, total % _NS
    nb = per + (s < rem).astype(jnp.int32)
    off = s * per + jnp.minimum(s, rem)
    return nb, off


def _deg_pass(dst2d):
    @functools.partial(
        pl.kernel,
        out_type=jax.ShapeDtypeStruct((_NC * _NPAD1,), jnp.float32),
        mesh=_mesh(),
        compiler_params=pltpu.CompilerParams(use_tc_tiling_on_sc=False),
        scratch_types=[
            pltpu.VMEM((_KB, _LN), jnp.int32),
            pltpu.VMEM((_LN,), jnp.float32),
            pltpu.VMEM((_ST1,), jnp.float32),
            pltpu.VMEM_SHARED((_NPAD1,), jnp.float32),
            pltpu.SemaphoreType.DMA,
        ],
    )
    def deg_kernel(dst_hbm, out_hbm, didx, ones_v, zb, acc, sem):
        c = lax.axis_index("c")
        s = lax.axis_index("s")

        def fill_ones(j, carry):
            ones_v[pl.ds(j * 16, 16)] = jnp.ones((16,), jnp.float32)
            return carry
        lax.fori_loop(0, _LN // 16, fill_ones, 0)

        def fill_z(j, carry):
            zb[pl.ds(j * 16, 16)] = jnp.zeros((16,), jnp.float32)
            return carry
        lax.fori_loop(0, _ST1 // 16, fill_z, 0)
        pltpu.sync_copy(zb, acc.at[pl.ds(s * _ST1, _ST1)])
        plsc.subcore_barrier()

        tot = _ROWS // _KB // _NC
        nb, off = _tile_range(tot, s)
        start = c * tot + off

        def macro(i, carry):
            b0 = (start + i) * _KB
            pltpu.sync_copy(dst_hbm.at[pl.ds(b0, _KB)], didx)
            hs = [pltpu.async_copy(ones_v, acc.at[didx.at[j]], sem, add=True)
                  for j in range(_KB)]
            for h in hs:
                h.wait()
            return carry
        lax.fori_loop(0, nb, macro, 0)

        plsc.subcore_barrier()
        pltpu.sync_copy(acc.at[pl.ds(s * _ST1, _ST1)], zb)
        pltpu.sync_copy(zb, out_hbm.at[pl.ds(c * _NPAD1 + s * _ST1, _ST1)])

    return deg_kernel(dst2d)


def _edge_pass(split_edges, t0, t1, src2d, dst2d):
    @functools.partial(
        pl.kernel,
        out_type=jax.ShapeDtypeStruct((_NC, _NPAD2, 16), jnp.float32),
        mesh=_mesh(),
        compiler_params=pltpu.CompilerParams(use_tc_tiling_on_sc=False),
        scratch_types=[
            pltpu.VMEM((2, _KB, _LN), jnp.int32),
            pltpu.VMEM((2, _KB, _LN), jnp.int32),
            pltpu.VMEM((_KB, _LN, 16), jnp.float32),
            pltpu.VMEM((_CK2, 16), jnp.float32),
            pltpu.VMEM_SHARED((_NPAD2, 16), jnp.float32),
            pltpu.SemaphoreType.DMA,
            pltpu.SemaphoreType.DMA,
            pltpu.SemaphoreType.DMA,
            pltpu.SemaphoreType.DMA,
        ],
    )
    def edge_kernel(t0_hbm, t1_hbm, src_hbm, dst_hbm, out_hbm,
                    sidx, didx, rows, stage, acc, isem0, isem1, gsem, ssem):
        c = lax.axis_index("c")
        s = lax.axis_index("s")
        isem = (isem0, isem1)

        def fill_z(j, carry):
            stage[j] = jnp.zeros((16,), jnp.float32)
            return carry
        lax.fori_loop(0, _CK2, fill_z, 0)

        def zc(t, carry):
            pltpu.sync_copy(stage, acc.at[pl.ds(s * _ST2 + t * _CK2, _CK2)])
            return carry
        lax.fori_loop(0, _NCK2, zc, 0)
        plsc.subcore_barrier()

        if split_edges:
            tot = _ROWS // _KB // _NC
            nb, off = _tile_range(tot, s)
            start = c * tot + off
        else:
            tot = _ROWS // _KB
            nb, off = _tile_range(tot, s)
            start = off

        def run(table):
            def fire_idx(b, i):
                r0 = (start + i) * _KB
                pltpu.async_copy(src_hbm.at[pl.ds(r0, _KB)], sidx.at[b],
                                 isem[b])
                pltpu.async_copy(dst_hbm.at[pl.ds(r0, _KB)], didx.at[b],
                                 isem[b])

            def wait_idx(b, i):
                r0 = (start + i) * _KB
                pltpu.make_async_copy(src_hbm.at[pl.ds(r0, _KB)], sidx.at[b],
                                      isem[b]).wait()
                pltpu.make_async_copy(dst_hbm.at[pl.ds(r0, _KB)], didx.at[b],
                                      isem[b]).wait()

            def stream_batch(b):
                for j in range(_KB):
                    pltpu.async_copy(table.at[sidx.at[b, j]], rows.at[j],
                                     gsem)
                for j in range(_KB):
                    pltpu.make_async_copy(table.at[sidx.at[b, j]],
                                          rows.at[j], gsem).wait()
                    pltpu.async_copy(rows.at[j], acc.at[didx.at[b, j]],
                                     ssem, add=True)
                for j in range(_KB):
                    pltpu.make_async_copy(rows.at[j], acc.at[didx.at[b, j]],
                                          ssem).wait()

            npair = nb // 2
            fire_idx(0, 0)

            def macro(p, carry):
                i0 = 2 * p
                wait_idx(0, i0)
                fire_idx(1, i0 + 1)
                stream_batch(0)
                wait_idx(1, i0 + 1)

                @pl.when(i0 + 2 < nb)
                def _():
                    fire_idx(0, i0 + 2)
                stream_batch(1)
                return carry
            lax.fori_loop(0, npair, macro, 0)

            @pl.when(nb % 2 == 1)
            def _():
                wait_idx(0, nb - 1)
                stream_batch(0)

        @pl.when(c == 0)
        def _():
            run(t0_hbm)

        @pl.when(c == 1)
        def _():
            run(t1_hbm)

        plsc.subcore_barrier()

        def rb(t, carry):
            r0 = s * _ST2 + t * _CK2
            pltpu.sync_copy(acc.at[pl.ds(r0, _CK2)], stage)
            pltpu.sync_copy(stage, out_hbm.at[c, pl.ds(r0, _CK2)])
            return carry
        lax.fori_loop(0, _NCK2, rb, 0)

    return edge_kernel(t0, t1, src2d, dst2d)


_BN = 5000


def _stage_b(d0, d1, x, W1):
    def body(d0_ref, d1_ref, x_ref, w_ref, y1_ref, dis_ref):
        deg = d0_ref[...] + d1_ref[...] + 1.0
        dis = lax.rsqrt(deg)
        y = jnp.dot(x_ref[...], w_ref[...], preferred_element_type=jnp.float32)
        y1_ref[...] = dis * y
        dis_ref[...] = dis

    return pl.pallas_call(
        body,
        grid=(_N // _BN,),
        in_specs=[
            pl.BlockSpec((_BN, 1), lambda i: (i, 0)),
            pl.BlockSpec((_BN, 1), lambda i: (i, 0)),
            pl.BlockSpec((_BN, 10), lambda i: (i, 0)),
            pl.BlockSpec((10, 16), lambda i: (0, 0)),
        ],
        out_specs=[
            pl.BlockSpec((_BN, 16), lambda i: (i, 0)),
            pl.BlockSpec((_BN, 1), lambda i: (i, 0)),
        ],
        out_shape=[
            jax.ShapeDtypeStruct((_N, 16), jnp.float32),
            jax.ShapeDtypeStruct((_N, 1), jnp.float32),
        ],
    )(d0, d1, x, W1)


def _stage_d(agg1, y1, dis, b1r):
    def body(agg_ref, y1_ref, dis_ref, b_ref, g_ref):
        tot = agg_ref[0] + agg_ref[1] + y1_ref[...]
        dis = dis_ref[...]
        h = jnp.maximum(dis * tot + b_ref[...], 0.0)
        g_ref[...] = dis * h

    return pl.pallas_call(
        body,
        grid=(_N // _BN,),
        in_specs=[
            pl.BlockSpec((_NC, _BN, 16), lambda i: (0, i, 0)),
            pl.BlockSpec((_BN, 16), lambda i: (i, 0)),
            pl.BlockSpec((_BN, 1), lambda i: (i, 0)),
            pl.BlockSpec((1, 16), lambda i: (0, 0)),
        ],
        out_specs=pl.BlockSpec((_BN, 16), lambda i: (i, 0)),
        out_shape=jax.ShapeDtypeStruct((_N, 16), jnp.float32),
    )(agg1, y1, dis, b1r)


def _stage_f(agg2, g, dis, W2, b2r, Wf1, bf1r, Wf2, bf2r):
    nblk = _N // _BN

    def body(agg_ref, g_ref, dis_ref, w2_ref, b2_ref,
             wf1_ref, bf1_ref, wf2_ref, bf2_ref, out_ref, acc_ref):
        i = pl.program_id(0)

        @pl.when(i == 0)
        def _():
            acc_ref[...] = jnp.zeros((1, 32), jnp.float32)
            out_ref[...] = jnp.zeros((1, 2), jnp.float32)

        tot = agg_ref[0] + agg_ref[1] + g_ref[...]
        y2 = jnp.dot(tot, w2_ref[...], preferred_element_type=jnp.float32)
        h = jnp.maximum(dis_ref[...] * y2 + b2_ref[...], 0.0)
        acc_ref[...] += jnp.sum(h, axis=0, keepdims=True)

        @pl.when(i == nblk - 1)
        def _():
            p = acc_ref[...] * (1.0 / _N)
            t = jnp.maximum(
                jnp.dot(p, wf1_ref[...], preferred_element_type=jnp.float32)
                + bf1_ref[...], 0.0)
            out_ref[...] = (jnp.dot(t, wf2_ref[...],
                                    preferred_element_type=jnp.float32)
                            + bf2_ref[...])

    return pl.pallas_call(
        body,
        grid=(nblk,),
        in_specs=[
            pl.BlockSpec((_NC, _BN, 16), lambda i: (0, i, 0)),
            pl.BlockSpec((_BN, 16), lambda i: (i, 0)),
            pl.BlockSpec((_BN, 1), lambda i: (i, 0)),
            pl.BlockSpec((16, 32), lambda i: (0, 0)),
            pl.BlockSpec((1, 32), lambda i: (0, 0)),
            pl.BlockSpec((32, 16), lambda i: (0, 0)),
            pl.BlockSpec((1, 16), lambda i: (0, 0)),
            pl.BlockSpec((16, 2), lambda i: (0, 0)),
            pl.BlockSpec((1, 2), lambda i: (0, 0)),
        ],
        out_specs=pl.BlockSpec((1, 2), lambda i: (0, 0)),
        out_shape=jax.ShapeDtypeStruct((1, 2), jnp.float32),
        scratch_shapes=[pltpu.VMEM((1, 32), jnp.float32)],
    )(agg2, g, dis, W2, b2r, Wf1, bf1r, Wf2, bf2r)


def kernel(x, edge_index, W1, b1, W2, b2, Wf1, bf1, Wf2, bf2):
    src2d = edge_index[0].reshape(_ROWS, _LN)
    dst2d = edge_index[1].reshape(_ROWS, _LN)

    deg2 = _deg_pass(dst2d)
    d0 = deg2[:_N].reshape(_N, 1)
    d1 = deg2[_NPAD1:_NPAD1 + _N].reshape(_N, 1)

    y1, dis = _stage_b(d0, d1, x, W1)
    agg1 = _edge_pass(True, y1, y1, src2d, dst2d)
    g = _stage_d(agg1, y1, dis, b1.reshape(1, 16))
    agg2 = _edge_pass(True, g, g, src2d, dst2d)
    return _stage_f(agg2, g, dis, W2, b2.reshape(1, 32),
                    Wf1, bf1.reshape(1, 16), Wf2, bf2.reshape(1, 2))


# speedup vs baseline: 1.0569x; 1.0569x over previous
import functools

import jax
import jax.numpy as jnp
from jax import lax
from jax.experimental import pallas as pl
from jax.experimental.pallas import tpu as pltpu
from jax.experimental.pallas import tpu_sc as plsc

_N = 100000
_E = 6400000
_LN = 128
_ROWS = _E // _LN
_KB = 10
_NC = 2
_NS = 16
_NPAD1 = 100096
_ST1 = _NPAD1 // _NS
_NPAD2 = 100096
_ST2 = _NPAD2 // _NS
_CK2 = 184
_NCK2 = _ST2 // _CK2


def _mesh():
    return plsc.VectorSubcoreMesh(
        core_axis_name="c", subcore_axis_name="s",
        num_cores=_NC, num_subcores=_NS)


def _tile_range(total, s):
    per, rem = total // _NS, total % _NS
    nb = per + (s < rem).astype(jnp.int32)
    off = s * per + jnp.minimum(s, rem)
    return nb, off


def _deg_pass(dst2d):
    @functools.partial(
        pl.kernel,
        out_type=jax.ShapeDtypeStruct((_NC * _NPAD1,), jnp.float32),
        mesh=_mesh(),
        compiler_params=pltpu.CompilerParams(use_tc_tiling_on_sc=False),
        scratch_types=[
            pltpu.VMEM((2, _KB, _LN), jnp.int32),
            pltpu.VMEM((_LN,), jnp.float32),
            pltpu.VMEM((_ST1,), jnp.float32),
            pltpu.VMEM_SHARED((_NPAD1,), jnp.float32),
            pltpu.SemaphoreType.DMA,
            pltpu.SemaphoreType.DMA,
            pltpu.SemaphoreType.DMA,
        ],
    )
    def deg_kernel(dst_hbm, out_hbm, didx, ones_v, zb, acc, isem0, isem1, sem):
        c = lax.axis_index("c")
        s = lax.axis_index("s")
        isem = (isem0, isem1)

        def fill_ones(j, carry):
            ones_v[pl.ds(j * 16, 16)] = jnp.ones((16,), jnp.float32)
            return carry
        lax.fori_loop(0, _LN // 16, fill_ones, 0)

        def fill_z(j, carry):
            zb[pl.ds(j * 16, 16)] = jnp.zeros((16,), jnp.float32)
            return carry
        lax.fori_loop(0, _ST1 // 16, fill_z, 0)
        pltpu.sync_copy(zb, acc.at[pl.ds(s * _ST1, _ST1)])
        plsc.subcore_barrier()

        tot = _ROWS // _KB // _NC
        nb, off = _tile_range(tot, s)
        start = c * tot + off

        def fire_idx(b, i):
            r0 = (start + i) * _KB
            pltpu.async_copy(dst_hbm.at[pl.ds(r0, _KB)], didx.at[b], isem[b])

        def wait_idx(b, i):
            r0 = (start + i) * _KB
            pltpu.make_async_copy(dst_hbm.at[pl.ds(r0, _KB)], didx.at[b],
                                  isem[b]).wait()

        def scatter_batch(b):
            for j in range(_KB):
                pltpu.async_copy(ones_v, acc.at[didx.at[b, j]], sem, add=True)
            for j in range(_KB):
                pltpu.make_async_copy(ones_v, acc.at[didx.at[b, j]],
                                      sem).wait()

        npair = nb // 2
        fire_idx(0, 0)

        def macro(p, carry):
            i0 = 2 * p
            wait_idx(0, i0)
            fire_idx(1, i0 + 1)
            scatter_batch(0)
            wait_idx(1, i0 + 1)

            @pl.when(i0 + 2 < nb)
            def _():
                fire_idx(0, i0 + 2)
            scatter_batch(1)
            return carry
        lax.fori_loop(0, npair, macro, 0)

        @pl.when(nb % 2 == 1)
        def _():
            wait_idx(0, nb - 1)
            scatter_batch(0)

        plsc.subcore_barrier()
        pltpu.sync_copy(acc.at[pl.ds(s * _ST1, _ST1)], zb)
        pltpu.sync_copy(zb, out_hbm.at[pl.ds(c * _NPAD1 + s * _ST1, _ST1)])

    return deg_kernel(dst2d)


def _edge_pass(split_edges, t0, t1, src2d, dst2d):
    @functools.partial(
        pl.kernel,
        out_type=jax.ShapeDtypeStruct((_NC, _NPAD2, 16), jnp.float32),
        mesh=_mesh(),
        compiler_params=pltpu.CompilerParams(use_tc_tiling_on_sc=False),
        scratch_types=[
            pltpu.VMEM((2, _KB, _LN), jnp.int32),
            pltpu.VMEM((2, _KB, _LN), jnp.int32),
            pltpu.VMEM((_KB, _LN, 16), jnp.float32),
            pltpu.VMEM((_CK2, 16), jnp.float32),
            pltpu.VMEM_SHARED((_NPAD2, 16), jnp.float32),
            pltpu.SemaphoreType.DMA,
            pltpu.SemaphoreType.DMA,
            pltpu.SemaphoreType.DMA,
            pltpu.SemaphoreType.DMA,
        ],
    )
    def edge_kernel(t0_hbm, t1_hbm, src_hbm, dst_hbm, out_hbm,
                    sidx, didx, rows, stage, acc, isem0, isem1, gsem, ssem):
        c = lax.axis_index("c")
        s = lax.axis_index("s")
        isem = (isem0, isem1)

        def fill_z(j, carry):
            stage[j] = jnp.zeros((16,), jnp.float32)
            return carry
        lax.fori_loop(0, _CK2, fill_z, 0)

        def zc(t, carry):
            pltpu.sync_copy(stage, acc.at[pl.ds(s * _ST2 + t * _CK2, _CK2)])
            return carry
        lax.fori_loop(0, _NCK2, zc, 0)
        plsc.subcore_barrier()

        if split_edges:
            tot = _ROWS // _KB // _NC
            nb, off = _tile_range(tot, s)
            start = c * tot + off
        else:
            tot = _ROWS // _KB
            nb, off = _tile_range(tot, s)
            start = off

        def run(table):
            def fire_idx(b, i):
                r0 = (start + i) * _KB
                pltpu.async_copy(src_hbm.at[pl.ds(r0, _KB)], sidx.at[b],
                                 isem[b])
                pltpu.async_copy(dst_hbm.at[pl.ds(r0, _KB)], didx.at[b],
                                 isem[b])

            def wait_idx(b, i):
                r0 = (start + i) * _KB
                pltpu.make_async_copy(src_hbm.at[pl.ds(r0, _KB)], sidx.at[b],
                                      isem[b]).wait()
                pltpu.make_async_copy(dst_hbm.at[pl.ds(r0, _KB)], didx.at[b],
                                      isem[b]).wait()

            def stream_batch(b):
                for j in range(_KB):
                    pltpu.async_copy(table.at[sidx.at[b, j]], rows.at[j],
                                     gsem)
                for j in range(_KB):
                    pltpu.make_async_copy(table.at[sidx.at[b, j]],
                                          rows.at[j], gsem).wait()
                    pltpu.async_copy(rows.at[j], acc.at[didx.at[b, j]],
                                     ssem, add=True)
                for j in range(_KB):
                    pltpu.make_async_copy(rows.at[j], acc.at[didx.at[b, j]],
                                          ssem).wait()

            npair = nb // 2
            fire_idx(0, 0)

            def macro(p, carry):
                i0 = 2 * p
                wait_idx(0, i0)
                fire_idx(1, i0 + 1)
                stream_batch(0)
                wait_idx(1, i0 + 1)

                @pl.when(i0 + 2 < nb)
                def _():
                    fire_idx(0, i0 + 2)
                stream_batch(1)
                return carry
            lax.fori_loop(0, npair, macro, 0)

            @pl.when(nb % 2 == 1)
            def _():
                wait_idx(0, nb - 1)
                stream_batch(0)

        @pl.when(c == 0)
        def _():
            run(t0_hbm)

        @pl.when(c == 1)
        def _():
            run(t1_hbm)

        plsc.subcore_barrier()

        def rb(t, carry):
            r0 = s * _ST2 + t * _CK2
            pltpu.sync_copy(acc.at[pl.ds(r0, _CK2)], stage)
            pltpu.sync_copy(stage, out_hbm.at[c, pl.ds(r0, _CK2)])
            return carry
        lax.fori_loop(0, _NCK2, rb, 0)

    return edge_kernel(t0, t1, src2d, dst2d)


_BN = 5000


def _stage_b(d0, d1, x, W1):
    def body(d0_ref, d1_ref, x_ref, w_ref, y1_ref, dis_ref):
        deg = d0_ref[...] + d1_ref[...] + 1.0
        dis = lax.rsqrt(deg)
        y = jnp.dot(x_ref[...], w_ref[...], preferred_element_type=jnp.float32)
        y1_ref[...] = dis * y
        dis_ref[...] = dis

    return pl.pallas_call(
        body,
        grid=(_N // _BN,),
        in_specs=[
            pl.BlockSpec((_BN, 1), lambda i: (i, 0)),
            pl.BlockSpec((_BN, 1), lambda i: (i, 0)),
            pl.BlockSpec((_BN, 10), lambda i: (i, 0)),
            pl.BlockSpec((10, 16), lambda i: (0, 0)),
        ],
        out_specs=[
            pl.BlockSpec((_BN, 16), lambda i: (i, 0)),
            pl.BlockSpec((_BN, 1), lambda i: (i, 0)),
        ],
        out_shape=[
            jax.ShapeDtypeStruct((_N, 16), jnp.float32),
            jax.ShapeDtypeStruct((_N, 1), jnp.float32),
        ],
    )(d0, d1, x, W1)


def _stage_d(agg1, y1, dis, b1r):
    def body(agg_ref, y1_ref, dis_ref, b_ref, g_ref):
        tot = agg_ref[0] + agg_ref[1] + y1_ref[...]
        dis = dis_ref[...]
        h = jnp.maximum(dis * tot + b_ref[...], 0.0)
        g_ref[...] = dis * h

    return pl.pallas_call(
        body,
        grid=(_N // _BN,),
        in_specs=[
            pl.BlockSpec((_NC, _BN, 16), lambda i: (0, i, 0)),
            pl.BlockSpec((_BN, 16), lambda i: (i, 0)),
            pl.BlockSpec((_BN, 1), lambda i: (i, 0)),
            pl.BlockSpec((1, 16), lambda i: (0, 0)),
        ],
        out_specs=pl.BlockSpec((_BN, 16), lambda i: (i, 0)),
        out_shape=jax.ShapeDtypeStruct((_N, 16), jnp.float32),
    )(agg1, y1, dis, b1r)


def _stage_f(agg2, g, dis, W2, b2r, Wf1, bf1r, Wf2, bf2r):
    nblk = _N // _BN

    def body(agg_ref, g_ref, dis_ref, w2_ref, b2_ref,
             wf1_ref, bf1_ref, wf2_ref, bf2_ref, out_ref, acc_ref):
        i = pl.program_id(0)

        @pl.when(i == 0)
        def _():
            acc_ref[...] = jnp.zeros((1, 32), jnp.float32)
            out_ref[...] = jnp.zeros((1, 2), jnp.float32)

        tot = agg_ref[0] + agg_ref[1] + g_ref[...]
        y2 = jnp.dot(tot, w2_ref[...], preferred_element_type=jnp.float32)
        h = jnp.maximum(dis_ref[...] * y2 + b2_ref[...], 0.0)
        acc_ref[...] += jnp.sum(h, axis=0, keepdims=True)

        @pl.when(i == nblk - 1)
        def _():
            p = acc_ref[...] * (1.0 / _N)
            t = jnp.maximum(
                jnp.dot(p, wf1_ref[...], preferred_element_type=jnp.float32)
                + bf1_ref[...], 0.0)
            out_ref[...] = (jnp.dot(t, wf2_ref[...],
                                    preferred_element_type=jnp.float32)
                            + bf2_ref[...])

    return pl.pallas_call(
        body,
        grid=(nblk,),
        in_specs=[
            pl.BlockSpec((_NC, _BN, 16), lambda i: (0, i, 0)),
            pl.BlockSpec((_BN, 16), lambda i: (i, 0)),
            pl.BlockSpec((_BN, 1), lambda i: (i, 0)),
            pl.BlockSpec((16, 32), lambda i: (0, 0)),
            pl.BlockSpec((1, 32), lambda i: (0, 0)),
            pl.BlockSpec((32, 16), lambda i: (0, 0)),
            pl.BlockSpec((1, 16), lambda i: (0, 0)),
            pl.BlockSpec((16, 2), lambda i: (0, 0)),
            pl.BlockSpec((1, 2), lambda i: (0, 0)),
        ],
        out_specs=pl.BlockSpec((1, 2), lambda i: (0, 0)),
        out_shape=jax.ShapeDtypeStruct((1, 2), jnp.float32),
        scratch_shapes=[pltpu.VMEM((1, 32), jnp.float32)],
    )(agg2, g, dis, W2, b2r, Wf1, bf1r, Wf2, bf2r)


def kernel(x, edge_index, W1, b1, W2, b2, Wf1, bf1, Wf2, bf2):
    src2d = edge_index[0].reshape(_ROWS, _LN)
    dst2d = edge_index[1].reshape(_ROWS, _LN)

    deg2 = _deg_pass(dst2d)
    d0 = deg2[:_N].reshape(_N, 1)
    d1 = deg2[_NPAD1:_NPAD1 + _N].reshape(_N, 1)

    y1, dis = _stage_b(d0, d1, x, W1)
    agg1 = _edge_pass(True, y1, y1, src2d, dst2d)
    g = _stage_d(agg1, y1, dis, b1.reshape(1, 16))
    agg2 = _edge_pass(True, g, g, src2d, dst2d)
    return _stage_f(agg2, g, dis, W2, b2.reshape(1, 32),
                    Wf1, bf1.reshape(1, 16), Wf2, bf2.reshape(1, 2))
